# baseline (device time: 46606 ns/iter reference)
import os

import jax
import jax.numpy as jnp
from jax import lax
from jax.experimental import pallas as pl
from jax.experimental.pallas import tpu as pltpu

N_DEV = 8
N_PART = 4
_NO_COMM = os.environ.get("KERNEL_NO_COMM", "0") == "1"
_NO_COMPUTE = os.environ.get("KERNEL_NO_COMPUTE", "0") == "1"


def kernel(x, w_mat):
    m_per, k = x.shape
    n = w_mat.shape[1]
    n_per = n // N_DEV
    k_part = k // N_PART
    out_dtype = jnp.bfloat16

    order = list(range(1, N_DEV)) + [0]

    def body(x_ref, w_ref, out_ref, xv_ref, wv_ref, y_ref, x_sems, dma_sems,
             send_sems, recv_sems):
        my = lax.axis_index("i")

        def x_dma(p):
            sl = pl.ds(p * (m_per // 2), m_per // 2)
            return pltpu.make_async_copy(
                x_ref.at[sl, :], xv_ref.at[sl, :], x_sems.at[p])

        def w_dma(g, slot, p):
            col = lax.rem(my + g, N_DEV) * n_per
            rows = pl.ds(p * k_part, k_part)
            return pltpu.make_async_copy(
                w_ref.at[rows, pl.ds(col, n_per)],
                wv_ref.at[slot, rows, :],
                dma_sems.at[slot, p],
            )

        for p in range(2):
            x_dma(p).start()
        for i, g in enumerate(order):
            for p in range(N_PART):
                w_dma(g, i, p).start()
        for p in range(2):
            x_dma(p).wait()

        sends = []
        for i, g in enumerate(order):
            for p in range(N_PART):
                w_dma(g, i, p).wait()

            if _NO_COMPUTE:
                continue
            acc = jnp.dot(xv_ref[...], wv_ref[i],
                          preferred_element_type=jnp.float32)
            yc = jnp.maximum(acc, 0.0).astype(out_dtype)

            if g == 0:
                out_ref[pl.ds(my * m_per, m_per), :] = yc
            else:
                y_ref[g] = yc
                if _NO_COMM:
                    continue
                dst = lax.rem(my + g, N_DEV)
                rdma = pltpu.make_async_remote_copy(
                    src_ref=y_ref.at[g],
                    dst_ref=out_ref.at[pl.ds(my * m_per, m_per), :],
                    send_sem=send_sems.at[g],
                    recv_sem=recv_sems.at[g],
                    device_id=(dst,),
                    device_id_type=pl.DeviceIdType.MESH,
                )
                rdma.start()
                sends.append(rdma)

        if _NO_COMM or _NO_COMPUTE:
            return

        for g in range(1, N_DEV):
            src = lax.rem(my - g + N_DEV, N_DEV)
            recv = pltpu.make_async_remote_copy(
                src_ref=y_ref.at[g],
                dst_ref=out_ref.at[pl.ds(src * m_per, m_per), :],
                send_sem=send_sems.at[g],
                recv_sem=recv_sems.at[g],
                device_id=(src,),
                device_id_type=pl.DeviceIdType.MESH,
            )
            recv.wait_recv()

        for rdma in sends:
            rdma.wait_send()

    return pl.pallas_call(
        body,
        out_shape=jax.ShapeDtypeStruct((N_DEV * m_per, n_per), out_dtype),
        in_specs=[
            pl.BlockSpec(memory_space=pltpu.MemorySpace.HBM),
            pl.BlockSpec(memory_space=pltpu.MemorySpace.HBM),
        ],
        out_specs=pl.BlockSpec(memory_space=pltpu.VMEM),
        scratch_shapes=[
            pltpu.VMEM((m_per, k), jnp.float32),
            pltpu.VMEM((N_DEV, k, n_per), jnp.float32),
            pltpu.VMEM((N_DEV, m_per, n_per), out_dtype),
            pltpu.SemaphoreType.DMA((2,)),
            pltpu.SemaphoreType.DMA((N_DEV, N_PART)),
            pltpu.SemaphoreType.DMA((N_DEV,)),
            pltpu.SemaphoreType.DMA((N_DEV,)),
        ],
        compiler_params=pltpu.CompilerParams(
            vmem_limit_bytes=100 * 1024 * 1024,
        ),
    )(x, w_mat)


# device time: 31623 ns/iter; 1.4738x vs baseline; 1.4738x over previous
import os

import jax
import jax.numpy as jnp
from jax import lax
from jax.experimental import pallas as pl
from jax.experimental.pallas import tpu as pltpu

N_DEV = 8
N_PART = int(os.environ.get("KERNEL_NPART", "2"))
N_PAIR = 4
_NO_COMM = os.environ.get("KERNEL_NO_COMM", "0") == "1"
_NO_COMPUTE = os.environ.get("KERNEL_NO_COMPUTE", "0") == "1"


def kernel(x, w_mat):
    m_per, k = x.shape
    n = w_mat.shape[1]
    n_per = n // N_DEV
    k_part = k // N_PART
    out_dtype = jnp.bfloat16

    order = list(range(1, N_DEV)) + [0]

    def body(x_ref, w_ref, out_ref, xv_ref, wv_ref, y_ref, x_sems, dma_sems,
             send_sems, recv_sems):
        my = lax.axis_index("i")

        def x_dma(p):
            sl = pl.ds(p * (m_per // 2), m_per // 2)
            return pltpu.make_async_copy(
                x_ref.at[sl, :], xv_ref.at[sl, :], x_sems.at[p])

        for p in range(2):
            x_dma(p).start()

        barrier_sem = pltpu.get_barrier_semaphore()
        for peer in range(1, N_DEV):
            pl.semaphore_signal(
                barrier_sem, inc=1,
                device_id=(lax.rem(my + peer, N_DEV),),
                device_id_type=pl.DeviceIdType.MESH,
            )

        def w_dma(i, slot, p):
            col = lax.rem(my + order[i], N_DEV) * n_per
            rows = pl.ds(p * k_part, k_part)
            return pltpu.make_async_copy(
                w_ref.at[rows, pl.ds(col, n_per)],
                wv_ref.at[slot, rows, pl.ds((i % 2) * n_per, n_per)],
                dma_sems.at[slot, i % 2, p],
            )

        def start_pair(pair, slot):
            for i in (2 * pair, 2 * pair + 1):
                for p in range(N_PART):
                    w_dma(i, slot, p).start()

        def wait_pair(pair, slot):
            for i in (2 * pair, 2 * pair + 1):
                for p in range(N_PART):
                    w_dma(i, slot, p).wait()

        for pair in range(N_PAIR):
            start_pair(pair, pair)
        for p in range(2):
            x_dma(p).wait()

        sends = []
        barrier_done = False
        for pair in range(N_PAIR):
            slot = pair
            wait_pair(pair, slot)

            if _NO_COMPUTE:
                continue
            acc = jnp.dot(xv_ref[...], wv_ref[slot],
                          preferred_element_type=jnp.float32)
            yc = jnp.maximum(acc, 0.0).astype(out_dtype)

            for half in range(2):
                g = order[2 * pair + half]
                yh = yc[:, half * n_per:(half + 1) * n_per]
                if g == 0:
                    out_ref[pl.ds(my * m_per, m_per), :] = yh
                    continue
                y_ref[g] = yh
                if _NO_COMM:
                    continue
                if not barrier_done:
                    pl.semaphore_wait(barrier_sem, N_DEV - 1)
                    barrier_done = True
                dst = lax.rem(my + g, N_DEV)
                rdma = pltpu.make_async_remote_copy(
                    src_ref=y_ref.at[g],
                    dst_ref=out_ref.at[pl.ds(my * m_per, m_per), :],
                    send_sem=send_sems.at[g],
                    recv_sem=recv_sems.at[g],
                    device_id=(dst,),
                    device_id_type=pl.DeviceIdType.MESH,
                )
                rdma.start()
                sends.append(rdma)

        if _NO_COMM or _NO_COMPUTE:
            return

        for g in range(1, N_DEV):
            src = lax.rem(my - g + N_DEV, N_DEV)
            recv = pltpu.make_async_remote_copy(
                src_ref=y_ref.at[g],
                dst_ref=out_ref.at[pl.ds(src * m_per, m_per), :],
                send_sem=send_sems.at[g],
                recv_sem=recv_sems.at[g],
                device_id=(src,),
                device_id_type=pl.DeviceIdType.MESH,
            )
            recv.wait_recv()

        for rdma in sends:
            rdma.wait_send()

    return pl.pallas_call(
        body,
        out_shape=jax.ShapeDtypeStruct((N_DEV * m_per, n_per), out_dtype),
        in_specs=[
            pl.BlockSpec(memory_space=pltpu.MemorySpace.HBM),
            pl.BlockSpec(memory_space=pltpu.MemorySpace.HBM),
        ],
        out_specs=pl.BlockSpec(memory_space=pltpu.VMEM),
        scratch_shapes=[
            pltpu.VMEM((m_per, k), jnp.float32),
            pltpu.VMEM((N_PAIR, k, 2 * n_per), jnp.float32),
            pltpu.VMEM((N_DEV, m_per, n_per), out_dtype),
            pltpu.SemaphoreType.DMA((2,)),
            pltpu.SemaphoreType.DMA((N_PAIR, 2, N_PART)),
            pltpu.SemaphoreType.DMA((N_DEV,)),
            pltpu.SemaphoreType.DMA((N_DEV,)),
        ],
        compiler_params=pltpu.CompilerParams(
            vmem_limit_bytes=100 * 1024 * 1024,
            collective_id=0,
        ),
    )(x, w_mat)
